# baseline (speedup 1.0000x reference)
import functools

import jax
import jax.numpy as jnp
from jax import lax
from jax.experimental import pallas as pl
from jax.experimental.pallas import tpu as pltpu
from jax.experimental.pallas import tpu_sc as plsc

N = 50000
E = 800000
F_IN = 128
H = 64
C = 10
B = 128

NC = 2
NS = 16

E_PAD = 802816
CH_S = 392
CH_G = 28
CH_D = 196
ACC_ROWS = 50176
SPAN = ACC_ROWS // NS
LAST = N - (NS - 1) * SPAN
DEGW = 16

RB = 1000
GRID = N // RB

_f32 = jnp.float32


def _sc_mesh():
  return plsc.VectorSubcoreMesh(
      core_axis_name="c", subcore_axis_name="s", num_cores=NC, num_subcores=NS)


def _deg_sc(dst_d, ones_rows, zrows):
  @functools.partial(
      pl.kernel,
      out_type=jax.ShapeDtypeStruct((NC, N, DEGW), _f32),
      mesh=_sc_mesh(),
      compiler_params=pltpu.CompilerParams(use_tc_tiling_on_sc=False),
      scratch_types=[
          pltpu.VMEM_SHARED((ACC_ROWS, DEGW), _f32),
          pltpu.VMEM((CH_D, 128), jnp.int32),
          pltpu.VMEM((128, DEGW), _f32),
      ],
  )
  def k(dst_hbm, ones_hbm, z_hbm, out_hbm, acc_sh, dst_v, ones_v):
    c = lax.axis_index("c")
    s = lax.axis_index("s")
    pltpu.sync_copy(z_hbm, acc_sh.at[pl.ds(s * SPAN, SPAN)])
    pltpu.sync_copy(ones_hbm, ones_v)
    pltpu.sync_copy(dst_hbm.at[c * NS + s], dst_v)
    plsc.subcore_barrier()

    def body(j, carry):
      pltpu.sync_copy(ones_v, acc_sh.at[dst_v.at[j]], add=True)
      return carry

    lax.fori_loop(0, CH_D, body, 0)
    plsc.subcore_barrier()

    @pl.when(s < NS - 1)
    def _():
      pltpu.sync_copy(acc_sh.at[pl.ds(s * SPAN, SPAN)],
                      out_hbm.at[c].at[pl.ds(s * SPAN, SPAN)])

    @pl.when(s == NS - 1)
    def _():
      pltpu.sync_copy(acc_sh.at[pl.ds((NS - 1) * SPAN, LAST)],
                      out_hbm.at[c].at[pl.ds((NS - 1) * SPAN, LAST)])

  return k(dst_d, ones_rows, zrows)


def _agg_sc(h_planes, src_s, dst_s, zrows):
  @functools.partial(
      pl.kernel,
      out_type=jax.ShapeDtypeStruct((NC, N, 32), _f32),
      mesh=_sc_mesh(),
      compiler_params=pltpu.CompilerParams(use_tc_tiling_on_sc=False),
      scratch_types=[
          pltpu.VMEM_SHARED((ACC_ROWS, 32), _f32),
          pltpu.VMEM((CH_G, 128), jnp.int32),
          pltpu.VMEM((CH_G, 128), jnp.int32),
          [pltpu.VMEM((128, 32), _f32) for _ in range(4)],
          [pltpu.SemaphoreType.DMA for _ in range(4)],
          [pltpu.SemaphoreType.DMA for _ in range(4)],
      ],
  )
  def k(h_hbm, src_hbm, dst_hbm, z_hbm, out_hbm, acc_sh, src_v, dst_v, rows,
        gsems, ssems):
    c = lax.axis_index("c")
    s = lax.axis_index("s")
    pltpu.sync_copy(z_hbm, acc_sh.at[pl.ds(s * SPAN, SPAN)])
    plsc.subcore_barrier()
    hpl = h_hbm.at[c]

    def group(g, carry):
      pltpu.sync_copy(src_hbm.at[s].at[pl.ds(g * CH_G, CH_G)], src_v)
      pltpu.sync_copy(dst_hbm.at[s].at[pl.ds(g * CH_G, CH_G)], dst_v)

      def quad(q, carry2):
        j0 = 4 * q

        @pl.when(q > 0)
        def _():
          for k in range(4):
            pltpu.make_async_copy(
                rows[k], acc_sh.at[dst_v.at[j0 - 4 + k]], ssems[k]).wait()

        ds = [pltpu.async_copy(hpl.at[src_v.at[j0 + k]], rows[k], gsems[k])
              for k in range(4)]
        for k in range(4):
          ds[k].wait()
          pltpu.async_copy(rows[k], acc_sh.at[dst_v.at[j0 + k]], ssems[k],
                           add=True)
        return carry2

      lax.fori_loop(0, CH_G // 4, quad, 0)
      for k in range(4):
        pltpu.make_async_copy(
            rows[k], acc_sh.at[dst_v.at[CH_G - 4 + k]], ssems[k]).wait()
      return carry

    lax.fori_loop(0, CH_S // CH_G, group, 0)
    plsc.subcore_barrier()

    @pl.when(s < NS - 1)
    def _():
      pltpu.sync_copy(acc_sh.at[pl.ds(s * SPAN, SPAN)],
                      out_hbm.at[c].at[pl.ds(s * SPAN, SPAN)])

    @pl.when(s == NS - 1)
    def _():
      pltpu.sync_copy(acc_sh.at[pl.ds((NS - 1) * SPAN, LAST)],
                      out_hbm.at[c].at[pl.ds((NS - 1) * SPAN, LAST)])

  return k(h_planes, src_s, dst_s, zrows)


def _m1_body(x_ref, w_ref, degp_ref, h_ref, dinv_ref):
  deg = degp_ref[0, :, 0:1] + degp_ref[1, :, 0:1] + 1.0
  di = lax.rsqrt(deg)
  h = jnp.dot(x_ref[...], w_ref[...], preferred_element_type=_f32)
  hp = h * di
  h_ref[0] = hp[:, :32]
  h_ref[1] = hp[:, 32:]
  dinv_ref[...] = di


def _m1_tc(x, W1, degp):
  return pl.pallas_call(
      _m1_body,
      grid=(GRID,),
      in_specs=[
          pl.BlockSpec((RB, F_IN), lambda i: (i, 0)),
          pl.BlockSpec((F_IN, H), lambda i: (0, 0)),
          pl.BlockSpec((NC, RB, DEGW), lambda i: (0, i, 0)),
      ],
      out_specs=[
          pl.BlockSpec((NC, RB, 32), lambda i: (0, i, 0)),
          pl.BlockSpec((RB, 1), lambda i: (i, 0)),
      ],
      out_shape=[
          jax.ShapeDtypeStruct((NC, N, 32), _f32),
          jax.ShapeDtypeStruct((N, 1), _f32),
      ],
  )(x, W1, degp)


def _pm_body(acc_ref, h_ref, dinv_ref, b_ref, g_ref, be_ref, w_ref, hn_ref,
             stat_ref, weff_ref):
  i = pl.program_id(0)
  a = jnp.concatenate([acc_ref[0] + h_ref[0], acc_ref[1] + h_ref[1]], axis=1)
  v = jnp.maximum(a * dinv_ref[...] + b_ref[...], 0.0)

  @pl.when(i < GRID)
  def _():
    @pl.when(i == 0)
    def _():
      stat_ref[...] = jnp.zeros_like(stat_ref)

    stat_ref[0:1] += jnp.sum(v, axis=0, keepdims=True)
    stat_ref[1:2] += jnp.sum(v * v, axis=0, keepdims=True)

  @pl.when(i >= GRID)
  def _():
    @pl.when(i == GRID)
    def _():
      mu = stat_ref[0:1] / N
      var = stat_ref[1:2] / N - mu * mu
      scale = g_ref[...] / jnp.sqrt(var + 1e-5)
      shift = be_ref[...] - mu * scale
      weff_ref[0:H] = scale.reshape(H, 1) * w_ref[...]
      weff_ref[H:H + 1] = jnp.dot(shift, w_ref[...],
                                  preferred_element_type=_f32)

    h = (jnp.dot(v, weff_ref[0:H], preferred_element_type=_f32)
         + weff_ref[H:H + 1])
    hp = h * dinv_ref[...]
    hn_ref[0] = hp[:, :32]
    hn_ref[1] = hp[:, 32:]


def _pm_tc(acc, hpl, dinv, b_row, g_row, be_row, Wn):
  blk = lambda i: (jnp.where(i < GRID, i, i - GRID), 0)
  pblk = lambda i: (0, jnp.where(i < GRID, i, i - GRID), 0)
  hn, _, _ = pl.pallas_call(
      _pm_body,
      grid=(2 * GRID,),
      in_specs=[
          pl.BlockSpec((NC, RB, 32), pblk),
          pl.BlockSpec((NC, RB, 32), pblk),
          pl.BlockSpec((RB, 1), blk),
          pl.BlockSpec((1, H), lambda i: (0, 0)),
          pl.BlockSpec((1, H), lambda i: (0, 0)),
          pl.BlockSpec((1, H), lambda i: (0, 0)),
          pl.BlockSpec((H, H), lambda i: (0, 0)),
      ],
      out_specs=[
          pl.BlockSpec((NC, RB, 32), pblk),
          pl.BlockSpec((2, H), lambda i: (0, 0)),
          pl.BlockSpec((H + 1, H), lambda i: (0, 0)),
      ],
      out_shape=[
          jax.ShapeDtypeStruct((NC, N, 32), _f32),
          jax.ShapeDtypeStruct((2, H), _f32),
          jax.ShapeDtypeStruct((H + 1, H), _f32),
      ],
  )(acc, hpl, dinv, b_row, g_row, be_row, Wn)
  return hn


def _p4_body(acc_ref, h_ref, dinv_ref, b_ref, bt_ref, cs_ref, cq_ref, seg_ref,
             cnt_ref):
  a = jnp.concatenate([acc_ref[0] + h_ref[0], acc_ref[1] + h_ref[1]], axis=1)
  v = jnp.maximum(a * dinv_ref[...] + b_ref[...], 0.0)
  bt = bt_ref[0, 0, :]
  oh = (bt[:, None] == lax.broadcasted_iota(jnp.int32, (RB, B), 1)).astype(_f32)

  @pl.when(pl.program_id(0) == 0)
  def _():
    cs_ref[...] = jnp.zeros_like(cs_ref)
    cq_ref[...] = jnp.zeros_like(cq_ref)
    seg_ref[...] = jnp.zeros_like(seg_ref)
    cnt_ref[...] = jnp.zeros_like(cnt_ref)

  cs_ref[...] += jnp.sum(v, axis=0, keepdims=True)
  cq_ref[...] += jnp.sum(v * v, axis=0, keepdims=True)
  seg_ref[...] += lax.dot_general(oh, v, (((0,), (0,)), ((), ())),
                                  preferred_element_type=_f32)
  cnt_ref[...] += jnp.sum(oh, axis=0)[:, None]


def _p4_tc(acc, hpl, dinv, b_row, batch3):
  return pl.pallas_call(
      _p4_body,
      grid=(GRID,),
      in_specs=[
          pl.BlockSpec((NC, RB, 32), lambda i: (0, i, 0)),
          pl.BlockSpec((NC, RB, 32), lambda i: (0, i, 0)),
          pl.BlockSpec((RB, 1), lambda i: (i, 0)),
          pl.BlockSpec((1, H), lambda i: (0, 0)),
          pl.BlockSpec((1, 1, RB), lambda i: (i, 0, 0)),
      ],
      out_specs=[
          pl.BlockSpec((1, H), lambda i: (0, 0)),
          pl.BlockSpec((1, H), lambda i: (0, 0)),
          pl.BlockSpec((B, H), lambda i: (0, 0)),
          pl.BlockSpec((B, 1), lambda i: (0, 0)),
      ],
      out_shape=[
          jax.ShapeDtypeStruct((1, H), _f32),
          jax.ShapeDtypeStruct((1, H), _f32),
          jax.ShapeDtypeStruct((B, H), _f32),
          jax.ShapeDtypeStruct((B, 1), _f32),
      ],
  )(acc, hpl, dinv, b_row, batch3)


def _head_body(cs_ref, cq_ref, seg_ref, cnt_ref, g_ref, be_ref, fw_ref, fb_ref,
               out_ref):
  mu = cs_ref[...] / N
  var = cq_ref[...] / N - mu * mu
  scale = g_ref[...] / jnp.sqrt(var + 1e-5)
  pooled = seg_ref[...] / jnp.maximum(cnt_ref[...], 1.0)
  pbn = scale * (pooled - mu) + be_ref[...]
  logits = jnp.dot(pbn, fw_ref[...], preferred_element_type=_f32) + fb_ref[...]
  m = jnp.max(logits, axis=1, keepdims=True)
  e = jnp.exp(logits - m)
  out_ref[...] = logits - m - jnp.log(jnp.sum(e, axis=1, keepdims=True))


def _head_tc(cs, cq, seg, cnt, g_row, be_row, fcW, fcb_row):
  return pl.pallas_call(
      _head_body,
      out_shape=jax.ShapeDtypeStruct((B, C), _f32),
  )(cs, cq, seg, cnt, g_row, be_row, fcW, fcb_row)


def kernel(x, edge_index, batch, W1, b1, g1, be1, W2, b2, g2, be2, W3, b3, g3,
           be3, W4, b4, g4, be4, fcW, fcb):
  src = edge_index[0]
  dst = edge_index[1]
  padn = E_PAD - E
  src_p = jnp.concatenate([src, jnp.zeros((padn,), dtype=jnp.int32)])
  dst_p = jnp.concatenate(
      [dst, N + (jnp.arange(padn, dtype=jnp.int32) % (ACC_ROWS - N))])
  src_s = src_p.reshape(NS, CH_S, 128)
  dst_s = dst_p.reshape(NS, CH_S, 128)
  dst_d = dst_p.reshape(NC * NS, CH_D, 128)

  zrows32 = jnp.zeros((SPAN, 32), _f32)
  zrowsd = jnp.zeros((SPAN, DEGW), _f32)
  ones_rows = jnp.ones((128, DEGW), _f32)
  batch3 = batch.reshape(GRID, 1, RB)

  degp = _deg_sc(dst_d, ones_rows, zrowsd)

  hpl, dinv = _m1_tc(x, W1, degp)
  for (b_l, g_l, be_l, W_n) in ((b1, g1, be1, W2), (b2, g2, be2, W3),
                                (b3, g3, be3, W4)):
    acc = _agg_sc(hpl, src_s, dst_s, zrows32)
    hpl = _pm_tc(acc, hpl, dinv, b_l.reshape(1, H), g_l.reshape(1, H),
                 be_l.reshape(1, H), W_n)

  acc = _agg_sc(hpl, src_s, dst_s, zrows32)
  cs, cq, seg, cnt = _p4_tc(acc, hpl, dinv, b4.reshape(1, H), batch3)
  return _head_tc(cs, cq, seg, cnt, g4.reshape(1, H), be4.reshape(1, H), fcW,
                  fcb.reshape(1, C))


# speedup vs baseline: 1.0809x; 1.0809x over previous
import functools

import jax
import jax.numpy as jnp
from jax import lax
from jax.experimental import pallas as pl
from jax.experimental.pallas import tpu as pltpu
from jax.experimental.pallas import tpu_sc as plsc

N = 50000
E = 800000
F_IN = 128
H = 64
C = 10
B = 128

NC = 2
NS = 16

E_PAD = 802816
CH_S = 392
CH_G = 28
CH_D = 196
ACC_ROWS = 50160
SPAN = ACC_ROWS // NS
LAST = N - (NS - 1) * SPAN
DEGW = 16

RB = 1000
GRID = N // RB

_f32 = jnp.float32


def _sc_mesh():
  return plsc.VectorSubcoreMesh(
      core_axis_name="c", subcore_axis_name="s", num_cores=NC, num_subcores=NS)


def _deg_sc(dst_d, ones_rows, zrows):
  @functools.partial(
      pl.kernel,
      out_type=jax.ShapeDtypeStruct((NC, N, DEGW), _f32),
      mesh=_sc_mesh(),
      compiler_params=pltpu.CompilerParams(use_tc_tiling_on_sc=False),
      scratch_types=[
          pltpu.VMEM_SHARED((ACC_ROWS, DEGW), _f32),
          pltpu.VMEM((CH_D, 128), jnp.int32),
          pltpu.VMEM((128, DEGW), _f32),
      ],
  )
  def k(dst_hbm, ones_hbm, z_hbm, out_hbm, acc_sh, dst_v, ones_v):
    c = lax.axis_index("c")
    s = lax.axis_index("s")
    pltpu.sync_copy(z_hbm, acc_sh.at[pl.ds(s * SPAN, SPAN)])
    pltpu.sync_copy(ones_hbm, ones_v)
    pltpu.sync_copy(dst_hbm.at[c * NS + s], dst_v)
    plsc.subcore_barrier()

    def body(j, carry):
      pltpu.sync_copy(ones_v, acc_sh.at[dst_v.at[j]], add=True)
      return carry

    lax.fori_loop(0, CH_D, body, 0)
    plsc.subcore_barrier()

    @pl.when(s < NS - 1)
    def _():
      pltpu.sync_copy(acc_sh.at[pl.ds(s * SPAN, SPAN)],
                      out_hbm.at[c].at[pl.ds(s * SPAN, SPAN)])

    @pl.when(s == NS - 1)
    def _():
      pltpu.sync_copy(acc_sh.at[pl.ds((NS - 1) * SPAN, LAST)],
                      out_hbm.at[c].at[pl.ds((NS - 1) * SPAN, LAST)])

  return k(dst_d, ones_rows, zrows)


def _agg_sc(h_planes, src_s, dst_s, zrows):
  @functools.partial(
      pl.kernel,
      out_type=jax.ShapeDtypeStruct((NC, N, 32), _f32),
      mesh=_sc_mesh(),
      compiler_params=pltpu.CompilerParams(use_tc_tiling_on_sc=False),
      scratch_types=[
          pltpu.VMEM_SHARED((ACC_ROWS, 32), _f32),
          [pltpu.VMEM((CH_G, 128), jnp.int32) for _ in range(2)],
          [pltpu.VMEM((CH_G, 128), jnp.int32) for _ in range(2)],
          [pltpu.VMEM((128, 32), _f32) for _ in range(4)],
          [pltpu.SemaphoreType.DMA for _ in range(4)],
          [pltpu.SemaphoreType.DMA for _ in range(4)],
          [pltpu.SemaphoreType.DMA for _ in range(2)],
      ],
  )
  def k(h_hbm, src_hbm, dst_hbm, z_hbm, out_hbm, acc_sh, srcs, dsts, rows,
        gsems, ssems, isems):
    c = lax.axis_index("c")
    s = lax.axis_index("s")
    hpl = h_hbm.at[c]
    my_src = src_hbm.at[s]
    my_dst = dst_hbm.at[s]

    def fire_idx(g, p):
      pltpu.async_copy(my_src.at[pl.ds(g * CH_G, CH_G)], srcs[p], isems[p])
      pltpu.async_copy(my_dst.at[pl.ds(g * CH_G, CH_G)], dsts[p], isems[p])

    def wait_idx(p):
      pltpu.make_async_copy(my_src.at[pl.ds(0, CH_G)], srcs[p],
                            isems[p]).wait()
      pltpu.make_async_copy(my_dst.at[pl.ds(0, CH_G)], dsts[p],
                            isems[p]).wait()

    fire_idx(0, 0)
    fire_idx(1, 1)
    pltpu.sync_copy(z_hbm, acc_sh.at[pl.ds(s * SPAN, SPAN)])
    plsc.subcore_barrier()

    def run_group(g, p):
      src_v = srcs[p]
      dst_v = dsts[p]
      wait_idx(p)

      def quad(q, carry2):
        j0 = 4 * q

        @pl.when(q > 0)
        def _():
          for k in range(4):
            pltpu.make_async_copy(
                rows[k], acc_sh.at[dst_v.at[j0 - 4 + k]], ssems[k]).wait()

        ds = [pltpu.async_copy(hpl.at[src_v.at[j0 + k]], rows[k], gsems[k])
              for k in range(4)]
        for k in range(4):
          ds[k].wait()
          pltpu.async_copy(rows[k], acc_sh.at[dst_v.at[j0 + k]], ssems[k],
                           add=True)
        return carry2

      lax.fori_loop(0, CH_G // 4, quad, 0)
      for k in range(4):
        pltpu.make_async_copy(
            rows[k], acc_sh.at[dst_v.at[CH_G - 4 + k]], ssems[k]).wait()

      @pl.when(g + 2 < CH_S // CH_G)
      def _():
        fire_idx(g + 2, p)

    def gpair(gg, carry):
      run_group(2 * gg, 0)
      run_group(2 * gg + 1, 1)
      return carry

    lax.fori_loop(0, CH_S // CH_G // 2, gpair, 0)
    plsc.subcore_barrier()

    @pl.when(s < NS - 1)
    def _():
      pltpu.sync_copy(acc_sh.at[pl.ds(s * SPAN, SPAN)],
                      out_hbm.at[c].at[pl.ds(s * SPAN, SPAN)])

    @pl.when(s == NS - 1)
    def _():
      pltpu.sync_copy(acc_sh.at[pl.ds((NS - 1) * SPAN, LAST)],
                      out_hbm.at[c].at[pl.ds((NS - 1) * SPAN, LAST)])

  return k(h_planes, src_s, dst_s, zrows)


def _m1_body(x_ref, w_ref, degp_ref, h_ref, dinv_ref):
  deg = degp_ref[0, :, 0:1] + degp_ref[1, :, 0:1] + 1.0
  di = lax.rsqrt(deg)
  h = jnp.dot(x_ref[...], w_ref[...], preferred_element_type=_f32)
  hp = h * di
  h_ref[0] = hp[:, :32]
  h_ref[1] = hp[:, 32:]
  dinv_ref[...] = di


def _m1_tc(x, W1, degp):
  return pl.pallas_call(
      _m1_body,
      grid=(GRID,),
      in_specs=[
          pl.BlockSpec((RB, F_IN), lambda i: (i, 0)),
          pl.BlockSpec((F_IN, H), lambda i: (0, 0)),
          pl.BlockSpec((NC, RB, DEGW), lambda i: (0, i, 0)),
      ],
      out_specs=[
          pl.BlockSpec((NC, RB, 32), lambda i: (0, i, 0)),
          pl.BlockSpec((RB, 1), lambda i: (i, 0)),
      ],
      out_shape=[
          jax.ShapeDtypeStruct((NC, N, 32), _f32),
          jax.ShapeDtypeStruct((N, 1), _f32),
      ],
  )(x, W1, degp)


def _m_body(v_ref, w_ref, c_ref, dinv_ref, h_ref):
  h = jnp.dot(v_ref[...], w_ref[...], preferred_element_type=_f32) + c_ref[...]
  hp = h * dinv_ref[...]
  h_ref[0] = hp[:, :32]
  h_ref[1] = hp[:, 32:]


def _m_tc(v, Weff, cshift, dinv):
  return pl.pallas_call(
      _m_body,
      grid=(GRID,),
      in_specs=[
          pl.BlockSpec((RB, H), lambda i: (i, 0)),
          pl.BlockSpec((H, H), lambda i: (0, 0)),
          pl.BlockSpec((1, H), lambda i: (0, 0)),
          pl.BlockSpec((RB, 1), lambda i: (i, 0)),
      ],
      out_specs=pl.BlockSpec((NC, RB, 32), lambda i: (0, i, 0)),
      out_shape=jax.ShapeDtypeStruct((NC, N, 32), _f32),
  )(v, Weff, cshift, dinv)


def _p_body(acc_ref, h_ref, dinv_ref, b_ref, v_ref, cs_ref, cq_ref):
  a = jnp.concatenate([acc_ref[0] + h_ref[0], acc_ref[1] + h_ref[1]], axis=1)
  v = jnp.maximum(a * dinv_ref[...] + b_ref[...], 0.0)
  v_ref[...] = v

  @pl.when(pl.program_id(0) == 0)
  def _():
    cs_ref[...] = jnp.zeros_like(cs_ref)
    cq_ref[...] = jnp.zeros_like(cq_ref)

  cs_ref[...] += jnp.sum(v, axis=0, keepdims=True)
  cq_ref[...] += jnp.sum(v * v, axis=0, keepdims=True)


def _p_tc(acc, hpl, dinv, b_row):
  return pl.pallas_call(
      _p_body,
      grid=(GRID,),
      in_specs=[
          pl.BlockSpec((NC, RB, 32), lambda i: (0, i, 0)),
          pl.BlockSpec((NC, RB, 32), lambda i: (0, i, 0)),
          pl.BlockSpec((RB, 1), lambda i: (i, 0)),
          pl.BlockSpec((1, H), lambda i: (0, 0)),
      ],
      out_specs=[
          pl.BlockSpec((RB, H), lambda i: (i, 0)),
          pl.BlockSpec((1, H), lambda i: (0, 0)),
          pl.BlockSpec((1, H), lambda i: (0, 0)),
      ],
      out_shape=[
          jax.ShapeDtypeStruct((N, H), _f32),
          jax.ShapeDtypeStruct((1, H), _f32),
          jax.ShapeDtypeStruct((1, H), _f32),
      ],
  )(acc, hpl, dinv, b_row)


def _p4_body(acc_ref, h_ref, dinv_ref, b_ref, bt_ref, cs_ref, cq_ref, seg_ref,
             cnt_ref):
  a = jnp.concatenate([acc_ref[0] + h_ref[0], acc_ref[1] + h_ref[1]], axis=1)
  v = jnp.maximum(a * dinv_ref[...] + b_ref[...], 0.0)
  bt = bt_ref[0, 0, :]
  oh = (bt[:, None] == lax.broadcasted_iota(jnp.int32, (RB, B), 1)).astype(_f32)

  @pl.when(pl.program_id(0) == 0)
  def _():
    cs_ref[...] = jnp.zeros_like(cs_ref)
    cq_ref[...] = jnp.zeros_like(cq_ref)
    seg_ref[...] = jnp.zeros_like(seg_ref)
    cnt_ref[...] = jnp.zeros_like(cnt_ref)

  cs_ref[...] += jnp.sum(v, axis=0, keepdims=True)
  cq_ref[...] += jnp.sum(v * v, axis=0, keepdims=True)
  seg_ref[...] += lax.dot_general(oh, v, (((0,), (0,)), ((), ())),
                                  preferred_element_type=_f32)
  cnt_ref[...] += jnp.sum(oh, axis=0)[:, None]


def _p4_tc(acc, hpl, dinv, b_row, batch3):
  return pl.pallas_call(
      _p4_body,
      grid=(GRID,),
      in_specs=[
          pl.BlockSpec((NC, RB, 32), lambda i: (0, i, 0)),
          pl.BlockSpec((NC, RB, 32), lambda i: (0, i, 0)),
          pl.BlockSpec((RB, 1), lambda i: (i, 0)),
          pl.BlockSpec((1, H), lambda i: (0, 0)),
          pl.BlockSpec((1, 1, RB), lambda i: (i, 0, 0)),
      ],
      out_specs=[
          pl.BlockSpec((1, H), lambda i: (0, 0)),
          pl.BlockSpec((1, H), lambda i: (0, 0)),
          pl.BlockSpec((B, H), lambda i: (0, 0)),
          pl.BlockSpec((B, 1), lambda i: (0, 0)),
      ],
      out_shape=[
          jax.ShapeDtypeStruct((1, H), _f32),
          jax.ShapeDtypeStruct((1, H), _f32),
          jax.ShapeDtypeStruct((B, H), _f32),
          jax.ShapeDtypeStruct((B, 1), _f32),
      ],
  )(acc, hpl, dinv, b_row, batch3)


def _head_body(cs_ref, cq_ref, seg_ref, cnt_ref, g_ref, be_ref, fw_ref, fb_ref,
               out_ref):
  mu = cs_ref[...] / N
  var = cq_ref[...] / N - mu * mu
  scale = g_ref[...] / jnp.sqrt(var + 1e-5)
  pooled = seg_ref[...] / jnp.maximum(cnt_ref[...], 1.0)
  pbn = scale * (pooled - mu) + be_ref[...]
  logits = jnp.dot(pbn, fw_ref[...], preferred_element_type=_f32) + fb_ref[...]
  m = jnp.max(logits, axis=1, keepdims=True)
  e = jnp.exp(logits - m)
  out_ref[...] = logits - m - jnp.log(jnp.sum(e, axis=1, keepdims=True))


def _head_tc(cs, cq, seg, cnt, g_row, be_row, fcW, fcb_row):
  return pl.pallas_call(
      _head_body,
      out_shape=jax.ShapeDtypeStruct((B, C), _f32),
  )(cs, cq, seg, cnt, g_row, be_row, fcW, fcb_row)


def _fold_bn(cs, cq, g, be, Wn):
  mu = cs[0] / N
  var = cq[0] / N - mu * mu
  scale = g / jnp.sqrt(var + 1e-5)
  shift = be - mu * scale
  return scale[:, None] * Wn, (shift @ Wn)[None, :]


def kernel(x, edge_index, batch, W1, b1, g1, be1, W2, b2, g2, be2, W3, b3, g3,
           be3, W4, b4, g4, be4, fcW, fcb):
  src = edge_index[0]
  dst = edge_index[1]
  padn = E_PAD - E
  src_p = jnp.concatenate([src, jnp.zeros((padn,), dtype=jnp.int32)])
  dst_p = jnp.concatenate(
      [dst, N + (jnp.arange(padn, dtype=jnp.int32) % (ACC_ROWS - N))])
  src_s = src_p.reshape(NS, CH_S, 128)
  dst_s = dst_p.reshape(NS, CH_S, 128)
  dst_d = dst_p.reshape(NC * NS, CH_D, 128)

  zrows32 = jnp.zeros((SPAN, 32), _f32)
  zrowsd = jnp.zeros((SPAN, DEGW), _f32)
  ones_rows = jnp.ones((128, DEGW), _f32)
  batch3 = batch.reshape(GRID, 1, RB)

  degp = _deg_sc(dst_d, ones_rows, zrowsd)

  hpl, dinv = _m1_tc(x, W1, degp)
  acc = _agg_sc(hpl, src_s, dst_s, zrows32)
  v, cs, cq = _p_tc(acc, hpl, dinv, b1.reshape(1, H))

  for (g_l, be_l, W_n, b_n) in ((g1, be1, W2, b2), (g2, be2, W3, b3),
                                (g3, be3, W4, b4)):
    Weff, cshift = _fold_bn(cs, cq, g_l, be_l, W_n)
    hpl = _m_tc(v, Weff, cshift, dinv)
    acc = _agg_sc(hpl, src_s, dst_s, zrows32)
    if W_n is W4:
      cs, cq, seg, cnt = _p4_tc(acc, hpl, dinv, b_n.reshape(1, H), batch3)
    else:
      v, cs, cq = _p_tc(acc, hpl, dinv, b_n.reshape(1, H))

  return _head_tc(cs, cq, seg, cnt, g4.reshape(1, H), be4.reshape(1, H), fcW,
                  fcb.reshape(1, C))


# speedup vs baseline: 1.0875x; 1.0060x over previous
import functools

import jax
import jax.numpy as jnp
from jax import lax
from jax.experimental import pallas as pl
from jax.experimental.pallas import tpu as pltpu
from jax.experimental.pallas import tpu_sc as plsc

N = 50000
E = 800000
F_IN = 128
H = 64
C = 10
B = 128

NC = 2
NS = 16

E_PAD = 802816
CH_S = 392
CH_G = 28
CH_D = 196
ACC_ROWS = 50160
SPAN = ACC_ROWS // NS
LAST = N - (NS - 1) * SPAN
DEGW = 16

RB = 1000
GRID = N // RB

_f32 = jnp.float32


def _sc_mesh():
  return plsc.VectorSubcoreMesh(
      core_axis_name="c", subcore_axis_name="s", num_cores=NC, num_subcores=NS)


def _deg_sc(dst_d, ones_rows, zrows):
  @functools.partial(
      pl.kernel,
      out_type=jax.ShapeDtypeStruct((NC, N, DEGW), _f32),
      mesh=_sc_mesh(),
      compiler_params=pltpu.CompilerParams(use_tc_tiling_on_sc=False),
      scratch_types=[
          pltpu.VMEM_SHARED((ACC_ROWS, DEGW), _f32),
          pltpu.VMEM((CH_D, 128), jnp.int32),
          pltpu.VMEM((128, DEGW), _f32),
          [pltpu.SemaphoreType.DMA for _ in range(4)],
      ],
  )
  def k(dst_hbm, ones_hbm, z_hbm, out_hbm, acc_sh, dst_v, ones_v, ssems):
    c = lax.axis_index("c")
    s = lax.axis_index("s")
    pltpu.sync_copy(z_hbm, acc_sh.at[pl.ds(s * SPAN, SPAN)])
    pltpu.sync_copy(ones_hbm, ones_v)
    pltpu.sync_copy(dst_hbm.at[c * NS + s], dst_v)
    plsc.subcore_barrier()

    def quad(q, carry):
      j0 = 4 * q

      @pl.when(q > 0)
      def _():
        for k in range(4):
          pltpu.make_async_copy(
              ones_v, acc_sh.at[dst_v.at[j0 - 4 + k]], ssems[k]).wait()

      for k in range(4):
        pltpu.async_copy(ones_v, acc_sh.at[dst_v.at[j0 + k]], ssems[k],
                         add=True)
      return carry

    lax.fori_loop(0, CH_D // 4, quad, 0)
    for k in range(4):
      pltpu.make_async_copy(
          ones_v, acc_sh.at[dst_v.at[CH_D - 4 + k]], ssems[k]).wait()
    plsc.subcore_barrier()

    @pl.when(s < NS - 1)
    def _():
      pltpu.sync_copy(acc_sh.at[pl.ds(s * SPAN, SPAN)],
                      out_hbm.at[c].at[pl.ds(s * SPAN, SPAN)])

    @pl.when(s == NS - 1)
    def _():
      pltpu.sync_copy(acc_sh.at[pl.ds((NS - 1) * SPAN, LAST)],
                      out_hbm.at[c].at[pl.ds((NS - 1) * SPAN, LAST)])

  return k(dst_d, ones_rows, zrows)


def _agg_sc(h_planes, src_s, dst_s, zrows):
  @functools.partial(
      pl.kernel,
      out_type=jax.ShapeDtypeStruct((NC, N, 32), _f32),
      mesh=_sc_mesh(),
      compiler_params=pltpu.CompilerParams(use_tc_tiling_on_sc=False),
      scratch_types=[
          pltpu.VMEM_SHARED((ACC_ROWS, 32), _f32),
          [pltpu.VMEM((CH_G, 128), jnp.int32) for _ in range(2)],
          [pltpu.VMEM((CH_G, 128), jnp.int32) for _ in range(2)],
          [pltpu.VMEM((128, 32), _f32) for _ in range(4)],
          [pltpu.SemaphoreType.DMA for _ in range(4)],
          [pltpu.SemaphoreType.DMA for _ in range(4)],
          [pltpu.SemaphoreType.DMA for _ in range(2)],
      ],
  )
  def k(h_hbm, src_hbm, dst_hbm, z_hbm, out_hbm, acc_sh, srcs, dsts, rows,
        gsems, ssems, isems):
    c = lax.axis_index("c")
    s = lax.axis_index("s")
    hpl = h_hbm.at[c]
    my_src = src_hbm.at[s]
    my_dst = dst_hbm.at[s]

    def fire_idx(g, p):
      pltpu.async_copy(my_src.at[pl.ds(g * CH_G, CH_G)], srcs[p], isems[p])
      pltpu.async_copy(my_dst.at[pl.ds(g * CH_G, CH_G)], dsts[p], isems[p])

    def wait_idx(p):
      pltpu.make_async_copy(my_src.at[pl.ds(0, CH_G)], srcs[p],
                            isems[p]).wait()
      pltpu.make_async_copy(my_dst.at[pl.ds(0, CH_G)], dsts[p],
                            isems[p]).wait()

    fire_idx(0, 0)
    fire_idx(1, 1)
    pltpu.sync_copy(z_hbm, acc_sh.at[pl.ds(s * SPAN, SPAN)])
    plsc.subcore_barrier()

    def run_group(g, p):
      src_v = srcs[p]
      dst_v = dsts[p]
      wait_idx(p)

      def quad(q, carry2):
        j0 = 4 * q

        @pl.when(q > 0)
        def _():
          for k in range(4):
            pltpu.make_async_copy(
                rows[k], acc_sh.at[dst_v.at[j0 - 4 + k]], ssems[k]).wait()

        ds = [pltpu.async_copy(hpl.at[src_v.at[j0 + k]], rows[k], gsems[k])
              for k in range(4)]
        for k in range(4):
          ds[k].wait()
          pltpu.async_copy(rows[k], acc_sh.at[dst_v.at[j0 + k]], ssems[k],
                           add=True)
        return carry2

      lax.fori_loop(0, CH_G // 4, quad, 0)
      for k in range(4):
        pltpu.make_async_copy(
            rows[k], acc_sh.at[dst_v.at[CH_G - 4 + k]], ssems[k]).wait()

      @pl.when(g + 2 < CH_S // CH_G)
      def _():
        fire_idx(g + 2, p)

    def gpair(gg, carry):
      run_group(2 * gg, 0)
      run_group(2 * gg + 1, 1)
      return carry

    lax.fori_loop(0, CH_S // CH_G // 2, gpair, 0)
    plsc.subcore_barrier()

    @pl.when(s < NS - 1)
    def _():
      pltpu.sync_copy(acc_sh.at[pl.ds(s * SPAN, SPAN)],
                      out_hbm.at[c].at[pl.ds(s * SPAN, SPAN)])

    @pl.when(s == NS - 1)
    def _():
      pltpu.sync_copy(acc_sh.at[pl.ds((NS - 1) * SPAN, LAST)],
                      out_hbm.at[c].at[pl.ds((NS - 1) * SPAN, LAST)])

  return k(h_planes, src_s, dst_s, zrows)


def _m1_body(x_ref, w_ref, degp_ref, h_ref, dinv_ref):
  deg = degp_ref[0, :, 0:1] + degp_ref[1, :, 0:1] + 1.0
  di = lax.rsqrt(deg)
  h = jnp.dot(x_ref[...], w_ref[...], preferred_element_type=_f32)
  hp = h * di
  h_ref[0] = hp[:, :32]
  h_ref[1] = hp[:, 32:]
  dinv_ref[...] = di


def _m1_tc(x, W1, degp):
  return pl.pallas_call(
      _m1_body,
      grid=(GRID,),
      in_specs=[
          pl.BlockSpec((RB, F_IN), lambda i: (i, 0)),
          pl.BlockSpec((F_IN, H), lambda i: (0, 0)),
          pl.BlockSpec((NC, RB, DEGW), lambda i: (0, i, 0)),
      ],
      out_specs=[
          pl.BlockSpec((NC, RB, 32), lambda i: (0, i, 0)),
          pl.BlockSpec((RB, 1), lambda i: (i, 0)),
      ],
      out_shape=[
          jax.ShapeDtypeStruct((NC, N, 32), _f32),
          jax.ShapeDtypeStruct((N, 1), _f32),
      ],
  )(x, W1, degp)


def _m_body(v_ref, w_ref, c_ref, dinv_ref, h_ref):
  h = jnp.dot(v_ref[...], w_ref[...], preferred_element_type=_f32) + c_ref[...]
  hp = h * dinv_ref[...]
  h_ref[0] = hp[:, :32]
  h_ref[1] = hp[:, 32:]


def _m_tc(v, Weff, cshift, dinv):
  return pl.pallas_call(
      _m_body,
      grid=(GRID,),
      in_specs=[
          pl.BlockSpec((RB, H), lambda i: (i, 0)),
          pl.BlockSpec((H, H), lambda i: (0, 0)),
          pl.BlockSpec((1, H), lambda i: (0, 0)),
          pl.BlockSpec((RB, 1), lambda i: (i, 0)),
      ],
      out_specs=pl.BlockSpec((NC, RB, 32), lambda i: (0, i, 0)),
      out_shape=jax.ShapeDtypeStruct((NC, N, 32), _f32),
  )(v, Weff, cshift, dinv)


def _p_body(acc_ref, h_ref, dinv_ref, b_ref, v_ref, cs_ref, cq_ref):
  a = jnp.concatenate([acc_ref[0] + h_ref[0], acc_ref[1] + h_ref[1]], axis=1)
  v = jnp.maximum(a * dinv_ref[...] + b_ref[...], 0.0)
  v_ref[...] = v

  @pl.when(pl.program_id(0) == 0)
  def _():
    cs_ref[...] = jnp.zeros_like(cs_ref)
    cq_ref[...] = jnp.zeros_like(cq_ref)

  cs_ref[...] += jnp.sum(v, axis=0, keepdims=True)
  cq_ref[...] += jnp.sum(v * v, axis=0, keepdims=True)


def _p_tc(acc, hpl, dinv, b_row):
  return pl.pallas_call(
      _p_body,
      grid=(GRID,),
      in_specs=[
          pl.BlockSpec((NC, RB, 32), lambda i: (0, i, 0)),
          pl.BlockSpec((NC, RB, 32), lambda i: (0, i, 0)),
          pl.BlockSpec((RB, 1), lambda i: (i, 0)),
          pl.BlockSpec((1, H), lambda i: (0, 0)),
      ],
      out_specs=[
          pl.BlockSpec((RB, H), lambda i: (i, 0)),
          pl.BlockSpec((1, H), lambda i: (0, 0)),
          pl.BlockSpec((1, H), lambda i: (0, 0)),
      ],
      out_shape=[
          jax.ShapeDtypeStruct((N, H), _f32),
          jax.ShapeDtypeStruct((1, H), _f32),
          jax.ShapeDtypeStruct((1, H), _f32),
      ],
  )(acc, hpl, dinv, b_row)


def _p4_body(acc_ref, h_ref, dinv_ref, b_ref, bt_ref, g_ref, be_ref, fw_ref,
             fb_ref, cs_ref, cq_ref, seg_ref, cnt_ref, out_ref):
  i = pl.program_id(0)

  @pl.when(i < GRID)
  def _():
    a = jnp.concatenate([acc_ref[0] + h_ref[0], acc_ref[1] + h_ref[1]], axis=1)
    v = jnp.maximum(a * dinv_ref[...] + b_ref[...], 0.0)
    bt = bt_ref[0, 0, :]
    oh = (bt[:, None] == lax.broadcasted_iota(jnp.int32,
                                              (RB, B), 1)).astype(_f32)

    @pl.when(i == 0)
    def _():
      cs_ref[...] = jnp.zeros_like(cs_ref)
      cq_ref[...] = jnp.zeros_like(cq_ref)
      seg_ref[...] = jnp.zeros_like(seg_ref)
      cnt_ref[...] = jnp.zeros_like(cnt_ref)

    cs_ref[...] += jnp.sum(v, axis=0, keepdims=True)
    cq_ref[...] += jnp.sum(v * v, axis=0, keepdims=True)
    seg_ref[...] += lax.dot_general(oh, v, (((0,), (0,)), ((), ())),
                                    preferred_element_type=_f32)
    cnt_ref[...] += jnp.sum(oh, axis=0)[:, None]

  @pl.when(i == GRID)
  def _():
    mu = cs_ref[...] / N
    var = cq_ref[...] / N - mu * mu
    scale = g_ref[...] / jnp.sqrt(var + 1e-5)
    pooled = seg_ref[...] / jnp.maximum(cnt_ref[...], 1.0)
    pbn = scale * (pooled - mu) + be_ref[...]
    logits = (jnp.dot(pbn, fw_ref[...], preferred_element_type=_f32)
              + fb_ref[...])
    m = jnp.max(logits, axis=1, keepdims=True)
    e = jnp.exp(logits - m)
    out_ref[...] = logits - m - jnp.log(jnp.sum(e, axis=1, keepdims=True))


def _p4_tc(acc, hpl, dinv, b_row, batch3, g_row, be_row, fcW, fcb_row):
  blk = lambda i: (jnp.minimum(i, GRID - 1), 0)
  pblk = lambda i: (0, jnp.minimum(i, GRID - 1), 0)
  outs = pl.pallas_call(
      _p4_body,
      grid=(GRID + 1,),
      in_specs=[
          pl.BlockSpec((NC, RB, 32), pblk),
          pl.BlockSpec((NC, RB, 32), pblk),
          pl.BlockSpec((RB, 1), blk),
          pl.BlockSpec((1, H), lambda i: (0, 0)),
          pl.BlockSpec((1, 1, RB), lambda i: (jnp.minimum(i, GRID - 1), 0, 0)),
          pl.BlockSpec((1, H), lambda i: (0, 0)),
          pl.BlockSpec((1, H), lambda i: (0, 0)),
          pl.BlockSpec((H, C), lambda i: (0, 0)),
          pl.BlockSpec((1, C), lambda i: (0, 0)),
      ],
      out_specs=[
          pl.BlockSpec((1, H), lambda i: (0, 0)),
          pl.BlockSpec((1, H), lambda i: (0, 0)),
          pl.BlockSpec((B, H), lambda i: (0, 0)),
          pl.BlockSpec((B, 1), lambda i: (0, 0)),
          pl.BlockSpec((B, C), lambda i: (0, 0)),
      ],
      out_shape=[
          jax.ShapeDtypeStruct((1, H), _f32),
          jax.ShapeDtypeStruct((1, H), _f32),
          jax.ShapeDtypeStruct((B, H), _f32),
          jax.ShapeDtypeStruct((B, 1), _f32),
          jax.ShapeDtypeStruct((B, C), _f32),
      ],
  )(acc, hpl, dinv, b_row, batch3, g_row, be_row, fcW, fcb_row)
  return outs[4]


def _fold_bn(cs, cq, g, be, Wn):
  mu = cs[0] / N
  var = cq[0] / N - mu * mu
  scale = g / jnp.sqrt(var + 1e-5)
  shift = be - mu * scale
  return scale[:, None] * Wn, (shift @ Wn)[None, :]


def kernel(x, edge_index, batch, W1, b1, g1, be1, W2, b2, g2, be2, W3, b3, g3,
           be3, W4, b4, g4, be4, fcW, fcb):
  src = edge_index[0]
  dst = edge_index[1]
  padn = E_PAD - E
  src_p = jnp.concatenate([src, jnp.zeros((padn,), dtype=jnp.int32)])
  dst_p = jnp.concatenate(
      [dst, N + (jnp.arange(padn, dtype=jnp.int32) % (ACC_ROWS - N))])
  src_s = src_p.reshape(NS, CH_S, 128)
  dst_s = dst_p.reshape(NS, CH_S, 128)
  dst_d = dst_p.reshape(NC * NS, CH_D, 128)

  zrows32 = jnp.zeros((SPAN, 32), _f32)
  zrowsd = jnp.zeros((SPAN, DEGW), _f32)
  ones_rows = jnp.ones((128, DEGW), _f32)
  batch3 = batch.reshape(GRID, 1, RB)

  degp = _deg_sc(dst_d, ones_rows, zrowsd)

  hpl, dinv = _m1_tc(x, W1, degp)
  acc = _agg_sc(hpl, src_s, dst_s, zrows32)
  v, cs, cq = _p_tc(acc, hpl, dinv, b1.reshape(1, H))

  for (g_l, be_l, W_n, b_n) in ((g1, be1, W2, b2), (g2, be2, W3, b3),
                                (g3, be3, W4, b4)):
    Weff, cshift = _fold_bn(cs, cq, g_l, be_l, W_n)
    hpl = _m_tc(v, Weff, cshift, dinv)
    acc = _agg_sc(hpl, src_s, dst_s, zrows32)
    if W_n is W4:
      return _p4_tc(acc, hpl, dinv, b_n.reshape(1, H), batch3,
                    g4.reshape(1, H), be4.reshape(1, H), fcW,
                    fcb.reshape(1, C))
    v, cs, cq = _p_tc(acc, hpl, dinv, b_n.reshape(1, H))


# speedup vs baseline: 1.1638x; 1.0702x over previous
import functools

import jax
import jax.numpy as jnp
from jax import lax
from jax.experimental import pallas as pl
from jax.experimental.pallas import tpu as pltpu
from jax.experimental.pallas import tpu_sc as plsc

N = 50000
E = 800000
F_IN = 128
H = 64
C = 10
B = 128

NC = 2
NS = 16

E_PAD = 802816
CH_S = 392
CH_G = 28
CH_D = 196
ACC_ROWS = 50160
SPAN = ACC_ROWS // NS
LAST = N - (NS - 1) * SPAN
DEGW = 16

RB = 2000
GRID = N // RB

_f32 = jnp.float32


def _sc_mesh():
  return plsc.VectorSubcoreMesh(
      core_axis_name="c", subcore_axis_name="s", num_cores=NC, num_subcores=NS)


def _deg_sc(dst_d, ones_rows, zrows):
  @functools.partial(
      pl.kernel,
      out_type=jax.ShapeDtypeStruct((NC, N, DEGW), _f32),
      mesh=_sc_mesh(),
      compiler_params=pltpu.CompilerParams(use_tc_tiling_on_sc=False),
      scratch_types=[
          pltpu.VMEM_SHARED((ACC_ROWS, DEGW), _f32),
          pltpu.VMEM((CH_D, 128), jnp.int32),
          pltpu.VMEM((128, DEGW), _f32),
          [pltpu.SemaphoreType.DMA for _ in range(4)],
      ],
  )
  def k(dst_hbm, ones_hbm, z_hbm, out_hbm, acc_sh, dst_v, ones_v, ssems):
    c = lax.axis_index("c")
    s = lax.axis_index("s")
    pltpu.sync_copy(z_hbm, acc_sh.at[pl.ds(s * SPAN, SPAN)])
    pltpu.sync_copy(ones_hbm, ones_v)
    pltpu.sync_copy(dst_hbm.at[c * NS + s], dst_v)
    plsc.subcore_barrier()

    def quad(q, carry):
      j0 = 4 * q

      @pl.when(q > 0)
      def _():
        for k in range(4):
          pltpu.make_async_copy(
              ones_v, acc_sh.at[dst_v.at[j0 - 4 + k]], ssems[k]).wait()

      for k in range(4):
        pltpu.async_copy(ones_v, acc_sh.at[dst_v.at[j0 + k]], ssems[k],
                         add=True)
      return carry

    lax.fori_loop(0, CH_D // 4, quad, 0)
    for k in range(4):
      pltpu.make_async_copy(
          ones_v, acc_sh.at[dst_v.at[CH_D - 4 + k]], ssems[k]).wait()
    plsc.subcore_barrier()

    @pl.when(s < NS - 1)
    def _():
      pltpu.sync_copy(acc_sh.at[pl.ds(s * SPAN, SPAN)],
                      out_hbm.at[c].at[pl.ds(s * SPAN, SPAN)])

    @pl.when(s == NS - 1)
    def _():
      pltpu.sync_copy(acc_sh.at[pl.ds((NS - 1) * SPAN, LAST)],
                      out_hbm.at[c].at[pl.ds((NS - 1) * SPAN, LAST)])

  return k(dst_d, ones_rows, zrows)


def _agg_sc(h_planes, src_s, dst_s, zrows):
  @functools.partial(
      pl.kernel,
      out_type=jax.ShapeDtypeStruct((NC, N, 32), _f32),
      mesh=_sc_mesh(),
      compiler_params=pltpu.CompilerParams(use_tc_tiling_on_sc=False),
      scratch_types=[
          pltpu.VMEM_SHARED((ACC_ROWS, 32), _f32),
          [pltpu.VMEM((CH_G, 128), jnp.int32) for _ in range(2)],
          [pltpu.VMEM((CH_G, 128), jnp.int32) for _ in range(2)],
          [pltpu.VMEM((128, 32), _f32) for _ in range(4)],
          [pltpu.SemaphoreType.DMA for _ in range(4)],
          [pltpu.SemaphoreType.DMA for _ in range(4)],
          [pltpu.SemaphoreType.DMA for _ in range(2)],
      ],
  )
  def k(h_hbm, src_hbm, dst_hbm, z_hbm, out_hbm, acc_sh, srcs, dsts, rows,
        gsems, ssems, isems):
    c = lax.axis_index("c")
    s = lax.axis_index("s")
    hpl = h_hbm.at[c]
    my_src = src_hbm.at[s]
    my_dst = dst_hbm.at[s]

    def fire_idx(g, p):
      pltpu.async_copy(my_src.at[pl.ds(g * CH_G, CH_G)], srcs[p], isems[p])
      pltpu.async_copy(my_dst.at[pl.ds(g * CH_G, CH_G)], dsts[p], isems[p])

    def wait_idx(p):
      pltpu.make_async_copy(my_src.at[pl.ds(0, CH_G)], srcs[p],
                            isems[p]).wait()
      pltpu.make_async_copy(my_dst.at[pl.ds(0, CH_G)], dsts[p],
                            isems[p]).wait()

    fire_idx(0, 0)
    fire_idx(1, 1)
    pltpu.sync_copy(z_hbm, acc_sh.at[pl.ds(s * SPAN, SPAN)])
    plsc.subcore_barrier()

    def run_group(g, p):
      src_v = srcs[p]
      dst_v = dsts[p]
      wait_idx(p)

      def quad(q, carry2):
        j0 = 4 * q

        @pl.when(q > 0)
        def _():
          for k in range(4):
            pltpu.make_async_copy(
                rows[k], acc_sh.at[dst_v.at[j0 - 4 + k]], ssems[k]).wait()

        ds = [pltpu.async_copy(hpl.at[src_v.at[j0 + k]], rows[k], gsems[k])
              for k in range(4)]
        for k in range(4):
          ds[k].wait()
          pltpu.async_copy(rows[k], acc_sh.at[dst_v.at[j0 + k]], ssems[k],
                           add=True)
        return carry2

      lax.fori_loop(0, CH_G // 4, quad, 0)
      for k in range(4):
        pltpu.make_async_copy(
            rows[k], acc_sh.at[dst_v.at[CH_G - 4 + k]], ssems[k]).wait()

      @pl.when(g + 2 < CH_S // CH_G)
      def _():
        fire_idx(g + 2, p)

    def gpair(gg, carry):
      run_group(2 * gg, 0)
      run_group(2 * gg + 1, 1)
      return carry

    lax.fori_loop(0, CH_S // CH_G // 2, gpair, 0)
    plsc.subcore_barrier()

    @pl.when(s < NS - 1)
    def _():
      pltpu.sync_copy(acc_sh.at[pl.ds(s * SPAN, SPAN)],
                      out_hbm.at[c].at[pl.ds(s * SPAN, SPAN)])

    @pl.when(s == NS - 1)
    def _():
      pltpu.sync_copy(acc_sh.at[pl.ds((NS - 1) * SPAN, LAST)],
                      out_hbm.at[c].at[pl.ds((NS - 1) * SPAN, LAST)])

  return k(h_planes, src_s, dst_s, zrows)


def _m1_body(x_ref, w_ref, degp_ref, h_ref, dinv_ref):
  deg = degp_ref[0, :, 0:1] + degp_ref[1, :, 0:1] + 1.0
  di = lax.rsqrt(deg)
  h = jnp.dot(x_ref[...], w_ref[...], preferred_element_type=_f32)
  hp = h * di
  h_ref[0] = hp[:, :32]
  h_ref[1] = hp[:, 32:]
  dinv_ref[...] = di


def _m1_tc(x, W1, degp):
  return pl.pallas_call(
      _m1_body,
      grid=(GRID,),
      in_specs=[
          pl.BlockSpec((RB, F_IN), lambda i: (i, 0)),
          pl.BlockSpec((F_IN, H), lambda i: (0, 0)),
          pl.BlockSpec((NC, RB, DEGW), lambda i: (0, i, 0)),
      ],
      out_specs=[
          pl.BlockSpec((NC, RB, 32), lambda i: (0, i, 0)),
          pl.BlockSpec((RB, 1), lambda i: (i, 0)),
      ],
      out_shape=[
          jax.ShapeDtypeStruct((NC, N, 32), _f32),
          jax.ShapeDtypeStruct((N, 1), _f32),
      ],
  )(x, W1, degp)


def _m_body(v_ref, w_ref, c_ref, dinv_ref, h_ref):
  h = jnp.dot(v_ref[...], w_ref[...], preferred_element_type=_f32) + c_ref[...]
  hp = h * dinv_ref[...]
  h_ref[0] = hp[:, :32]
  h_ref[1] = hp[:, 32:]


def _m_tc(v, Weff, cshift, dinv):
  return pl.pallas_call(
      _m_body,
      grid=(GRID,),
      in_specs=[
          pl.BlockSpec((RB, H), lambda i: (i, 0)),
          pl.BlockSpec((H, H), lambda i: (0, 0)),
          pl.BlockSpec((1, H), lambda i: (0, 0)),
          pl.BlockSpec((RB, 1), lambda i: (i, 0)),
      ],
      out_specs=pl.BlockSpec((NC, RB, 32), lambda i: (0, i, 0)),
      out_shape=jax.ShapeDtypeStruct((NC, N, 32), _f32),
  )(v, Weff, cshift, dinv)


def _p_body(acc_ref, h_ref, dinv_ref, b_ref, v_ref, cs_ref, cq_ref):
  a = jnp.concatenate([acc_ref[0] + h_ref[0], acc_ref[1] + h_ref[1]], axis=1)
  v = jnp.maximum(a * dinv_ref[...] + b_ref[...], 0.0)
  v_ref[...] = v

  @pl.when(pl.program_id(0) == 0)
  def _():
    cs_ref[...] = jnp.zeros_like(cs_ref)
    cq_ref[...] = jnp.zeros_like(cq_ref)

  cs_ref[...] += jnp.sum(v, axis=0, keepdims=True)
  cq_ref[...] += jnp.sum(v * v, axis=0, keepdims=True)


def _p_tc(acc, hpl, dinv, b_row):
  return pl.pallas_call(
      _p_body,
      grid=(GRID,),
      in_specs=[
          pl.BlockSpec((NC, RB, 32), lambda i: (0, i, 0)),
          pl.BlockSpec((NC, RB, 32), lambda i: (0, i, 0)),
          pl.BlockSpec((RB, 1), lambda i: (i, 0)),
          pl.BlockSpec((1, H), lambda i: (0, 0)),
      ],
      out_specs=[
          pl.BlockSpec((RB, H), lambda i: (i, 0)),
          pl.BlockSpec((1, H), lambda i: (0, 0)),
          pl.BlockSpec((1, H), lambda i: (0, 0)),
      ],
      out_shape=[
          jax.ShapeDtypeStruct((N, H), _f32),
          jax.ShapeDtypeStruct((1, H), _f32),
          jax.ShapeDtypeStruct((1, H), _f32),
      ],
  )(acc, hpl, dinv, b_row)


def _p4_body(acc_ref, h_ref, dinv_ref, b_ref, bt_ref, g_ref, be_ref, fw_ref,
             fb_ref, cs_ref, cq_ref, seg_ref, cnt_ref, out_ref):
  i = pl.program_id(0)

  @pl.when(i < GRID)
  def _():
    a = jnp.concatenate([acc_ref[0] + h_ref[0], acc_ref[1] + h_ref[1]], axis=1)
    v = jnp.maximum(a * dinv_ref[...] + b_ref[...], 0.0)
    bt = bt_ref[0, 0, :]
    oh = (bt[:, None] == lax.broadcasted_iota(jnp.int32,
                                              (RB, B), 1)).astype(_f32)

    @pl.when(i == 0)
    def _():
      cs_ref[...] = jnp.zeros_like(cs_ref)
      cq_ref[...] = jnp.zeros_like(cq_ref)
      seg_ref[...] = jnp.zeros_like(seg_ref)
      cnt_ref[...] = jnp.zeros_like(cnt_ref)

    cs_ref[...] += jnp.sum(v, axis=0, keepdims=True)
    cq_ref[...] += jnp.sum(v * v, axis=0, keepdims=True)
    seg_ref[...] += lax.dot_general(oh, v, (((0,), (0,)), ((), ())),
                                    preferred_element_type=_f32)
    cnt_ref[...] += jnp.sum(oh, axis=0)[:, None]

  @pl.when(i == GRID)
  def _():
    mu = cs_ref[...] / N
    var = cq_ref[...] / N - mu * mu
    scale = g_ref[...] / jnp.sqrt(var + 1e-5)
    pooled = seg_ref[...] / jnp.maximum(cnt_ref[...], 1.0)
    pbn = scale * (pooled - mu) + be_ref[...]
    logits = (jnp.dot(pbn, fw_ref[...], preferred_element_type=_f32)
              + fb_ref[...])
    m = jnp.max(logits, axis=1, keepdims=True)
    e = jnp.exp(logits - m)
    out_ref[...] = logits - m - jnp.log(jnp.sum(e, axis=1, keepdims=True))


def _p4_tc(acc, hpl, dinv, b_row, batch3, g_row, be_row, fcW, fcb_row):
  blk = lambda i: (jnp.minimum(i, GRID - 1), 0)
  pblk = lambda i: (0, jnp.minimum(i, GRID - 1), 0)
  outs = pl.pallas_call(
      _p4_body,
      grid=(GRID + 1,),
      in_specs=[
          pl.BlockSpec((NC, RB, 32), pblk),
          pl.BlockSpec((NC, RB, 32), pblk),
          pl.BlockSpec((RB, 1), blk),
          pl.BlockSpec((1, H), lambda i: (0, 0)),
          pl.BlockSpec((1, 1, RB), lambda i: (jnp.minimum(i, GRID - 1), 0, 0)),
          pl.BlockSpec((1, H), lambda i: (0, 0)),
          pl.BlockSpec((1, H), lambda i: (0, 0)),
          pl.BlockSpec((H, C), lambda i: (0, 0)),
          pl.BlockSpec((1, C), lambda i: (0, 0)),
      ],
      out_specs=[
          pl.BlockSpec((1, H), lambda i: (0, 0)),
          pl.BlockSpec((1, H), lambda i: (0, 0)),
          pl.BlockSpec((B, H), lambda i: (0, 0)),
          pl.BlockSpec((B, 1), lambda i: (0, 0)),
          pl.BlockSpec((B, C), lambda i: (0, 0)),
      ],
      out_shape=[
          jax.ShapeDtypeStruct((1, H), _f32),
          jax.ShapeDtypeStruct((1, H), _f32),
          jax.ShapeDtypeStruct((B, H), _f32),
          jax.ShapeDtypeStruct((B, 1), _f32),
          jax.ShapeDtypeStruct((B, C), _f32),
      ],
  )(acc, hpl, dinv, b_row, batch3, g_row, be_row, fcW, fcb_row)
  return outs[4]


def _fold_bn(cs, cq, g, be, Wn):
  mu = cs[0] / N
  var = cq[0] / N - mu * mu
  scale = g / jnp.sqrt(var + 1e-5)
  shift = be - mu * scale
  return scale[:, None] * Wn, (shift @ Wn)[None, :]


def kernel(x, edge_index, batch, W1, b1, g1, be1, W2, b2, g2, be2, W3, b3, g3,
           be3, W4, b4, g4, be4, fcW, fcb):
  src = edge_index[0]
  dst = edge_index[1]
  padn = E_PAD - E
  src_p = jnp.concatenate([src, jnp.zeros((padn,), dtype=jnp.int32)])
  dst_p = jnp.concatenate(
      [dst, N + (jnp.arange(padn, dtype=jnp.int32) % (ACC_ROWS - N))])
  src_s = src_p.reshape(NS, CH_S, 128)
  dst_s = dst_p.reshape(NS, CH_S, 128)
  dst_d = dst_p.reshape(NC * NS, CH_D, 128)

  zrows32 = jnp.zeros((SPAN, 32), _f32)
  zrowsd = jnp.zeros((SPAN, DEGW), _f32)
  ones_rows = jnp.ones((128, DEGW), _f32)
  batch3 = batch.reshape(GRID, 1, RB)

  degp = _deg_sc(dst_d, ones_rows, zrowsd)

  hpl, dinv = _m1_tc(x, W1, degp)
  acc = _agg_sc(hpl, src_s, dst_s, zrows32)
  v, cs, cq = _p_tc(acc, hpl, dinv, b1.reshape(1, H))

  for (g_l, be_l, W_n, b_n) in ((g1, be1, W2, b2), (g2, be2, W3, b3),
                                (g3, be3, W4, b4)):
    Weff, cshift = _fold_bn(cs, cq, g_l, be_l, W_n)
    hpl = _m_tc(v, Weff, cshift, dinv)
    acc = _agg_sc(hpl, src_s, dst_s, zrows32)
    if W_n is W4:
      return _p4_tc(acc, hpl, dinv, b_n.reshape(1, H), batch3,
                    g4.reshape(1, H), be4.reshape(1, H), fcW,
                    fcb.reshape(1, C))
    v, cs, cq = _p_tc(acc, hpl, dinv, b_n.reshape(1, H))


# speedup vs baseline: 1.1883x; 1.0211x over previous
import functools

import jax
import jax.numpy as jnp
from jax import lax
from jax.experimental import pallas as pl
from jax.experimental.pallas import tpu as pltpu
from jax.experimental.pallas import tpu_sc as plsc

N = 50000
E = 800000
F_IN = 128
H = 64
C = 10
B = 128

NC = 2
NS = 16

E_PAD = 802816
CH_S = 392
CH_G = 28
CH_D = 196
ACC_ROWS = 50160
SPAN = ACC_ROWS // NS
LAST = N - (NS - 1) * SPAN
DEGW = 16

RB = 5000
GRID = N // RB

_f32 = jnp.float32


def _sc_mesh():
  return plsc.VectorSubcoreMesh(
      core_axis_name="c", subcore_axis_name="s", num_cores=NC, num_subcores=NS)


def _deg_sc(dst_d, ones_rows, zrows):
  @functools.partial(
      pl.kernel,
      out_type=jax.ShapeDtypeStruct((NC, N, DEGW), _f32),
      mesh=_sc_mesh(),
      compiler_params=pltpu.CompilerParams(use_tc_tiling_on_sc=False),
      scratch_types=[
          pltpu.VMEM_SHARED((ACC_ROWS, DEGW), _f32),
          pltpu.VMEM((CH_D, 128), jnp.int32),
          pltpu.VMEM((128, DEGW), _f32),
          [pltpu.SemaphoreType.DMA for _ in range(4)],
      ],
  )
  def k(dst_hbm, ones_hbm, z_hbm, out_hbm, acc_sh, dst_v, ones_v, ssems):
    c = lax.axis_index("c")
    s = lax.axis_index("s")
    pltpu.sync_copy(z_hbm, acc_sh.at[pl.ds(s * SPAN, SPAN)])
    pltpu.sync_copy(ones_hbm, ones_v)
    pltpu.sync_copy(dst_hbm.at[c * NS + s], dst_v)
    plsc.subcore_barrier()

    def quad(q, carry):
      j0 = 4 * q

      @pl.when(q > 0)
      def _():
        for k in range(4):
          pltpu.make_async_copy(
              ones_v, acc_sh.at[dst_v.at[j0 - 4 + k]], ssems[k]).wait()

      for k in range(4):
        pltpu.async_copy(ones_v, acc_sh.at[dst_v.at[j0 + k]], ssems[k],
                         add=True)
      return carry

    lax.fori_loop(0, CH_D // 4, quad, 0)
    for k in range(4):
      pltpu.make_async_copy(
          ones_v, acc_sh.at[dst_v.at[CH_D - 4 + k]], ssems[k]).wait()
    plsc.subcore_barrier()

    @pl.when(s < NS - 1)
    def _():
      pltpu.sync_copy(acc_sh.at[pl.ds(s * SPAN, SPAN)],
                      out_hbm.at[c].at[pl.ds(s * SPAN, SPAN)])

    @pl.when(s == NS - 1)
    def _():
      pltpu.sync_copy(acc_sh.at[pl.ds((NS - 1) * SPAN, LAST)],
                      out_hbm.at[c].at[pl.ds((NS - 1) * SPAN, LAST)])

  return k(dst_d, ones_rows, zrows)


def _agg_sc(h_planes, src_s, dst_s, zrows):
  @functools.partial(
      pl.kernel,
      out_type=jax.ShapeDtypeStruct((NC, N, 32), _f32),
      mesh=_sc_mesh(),
      compiler_params=pltpu.CompilerParams(use_tc_tiling_on_sc=False),
      scratch_types=[
          pltpu.VMEM_SHARED((ACC_ROWS, 32), _f32),
          [pltpu.VMEM((CH_G, 128), jnp.int32) for _ in range(2)],
          [pltpu.VMEM((CH_G, 128), jnp.int32) for _ in range(2)],
          [pltpu.VMEM((128, 32), _f32) for _ in range(4)],
          [pltpu.SemaphoreType.DMA for _ in range(4)],
          [pltpu.SemaphoreType.DMA for _ in range(4)],
          [pltpu.SemaphoreType.DMA for _ in range(2)],
      ],
  )
  def k(h_hbm, src_hbm, dst_hbm, z_hbm, out_hbm, acc_sh, srcs, dsts, rows,
        gsems, ssems, isems):
    c = lax.axis_index("c")
    s = lax.axis_index("s")
    hpl = h_hbm.at[c]
    my_src = src_hbm.at[s]
    my_dst = dst_hbm.at[s]

    def fire_idx(g, p):
      pltpu.async_copy(my_src.at[pl.ds(g * CH_G, CH_G)], srcs[p], isems[p])
      pltpu.async_copy(my_dst.at[pl.ds(g * CH_G, CH_G)], dsts[p], isems[p])

    def wait_idx(p):
      pltpu.make_async_copy(my_src.at[pl.ds(0, CH_G)], srcs[p],
                            isems[p]).wait()
      pltpu.make_async_copy(my_dst.at[pl.ds(0, CH_G)], dsts[p],
                            isems[p]).wait()

    fire_idx(0, 0)
    fire_idx(1, 1)
    pltpu.sync_copy(z_hbm, acc_sh.at[pl.ds(s * SPAN, SPAN)])
    plsc.subcore_barrier()

    def run_group(g, p):
      src_v = srcs[p]
      dst_v = dsts[p]
      wait_idx(p)

      def quad(q, carry2):
        j0 = 4 * q

        @pl.when(q > 0)
        def _():
          for k in range(4):
            pltpu.make_async_copy(
                rows[k], acc_sh.at[dst_v.at[j0 - 4 + k]], ssems[k]).wait()

        ds = [pltpu.async_copy(hpl.at[src_v.at[j0 + k]], rows[k], gsems[k])
              for k in range(4)]
        for k in range(4):
          ds[k].wait()
          pltpu.async_copy(rows[k], acc_sh.at[dst_v.at[j0 + k]], ssems[k],
                           add=True)
        return carry2

      lax.fori_loop(0, CH_G // 4, quad, 0)
      for k in range(4):
        pltpu.make_async_copy(
            rows[k], acc_sh.at[dst_v.at[CH_G - 4 + k]], ssems[k]).wait()

      @pl.when(g + 2 < CH_S // CH_G)
      def _():
        fire_idx(g + 2, p)

    def gpair(gg, carry):
      run_group(2 * gg, 0)
      run_group(2 * gg + 1, 1)
      return carry

    lax.fori_loop(0, CH_S // CH_G // 2, gpair, 0)
    plsc.subcore_barrier()

    @pl.when(s < NS - 1)
    def _():
      pltpu.sync_copy(acc_sh.at[pl.ds(s * SPAN, SPAN)],
                      out_hbm.at[c].at[pl.ds(s * SPAN, SPAN)])

    @pl.when(s == NS - 1)
    def _():
      pltpu.sync_copy(acc_sh.at[pl.ds((NS - 1) * SPAN, LAST)],
                      out_hbm.at[c].at[pl.ds((NS - 1) * SPAN, LAST)])

  return k(h_planes, src_s, dst_s, zrows)


def _m1_body(x_ref, w_ref, degp_ref, h_ref, dinv_ref):
  deg = degp_ref[0, :, 0:1] + degp_ref[1, :, 0:1] + 1.0
  di = lax.rsqrt(deg)
  h = jnp.dot(x_ref[...], w_ref[...], preferred_element_type=_f32)
  hp = h * di
  h_ref[0] = hp[:, :32]
  h_ref[1] = hp[:, 32:]
  dinv_ref[...] = di


def _m1_tc(x, W1, degp):
  return pl.pallas_call(
      _m1_body,
      grid=(GRID,),
      in_specs=[
          pl.BlockSpec((RB, F_IN), lambda i: (i, 0)),
          pl.BlockSpec((F_IN, H), lambda i: (0, 0)),
          pl.BlockSpec((NC, RB, DEGW), lambda i: (0, i, 0)),
      ],
      out_specs=[
          pl.BlockSpec((NC, RB, 32), lambda i: (0, i, 0)),
          pl.BlockSpec((RB, 1), lambda i: (i, 0)),
      ],
      out_shape=[
          jax.ShapeDtypeStruct((NC, N, 32), _f32),
          jax.ShapeDtypeStruct((N, 1), _f32),
      ],
  )(x, W1, degp)


def _m_body(v_ref, w_ref, c_ref, dinv_ref, h_ref):
  h = jnp.dot(v_ref[...], w_ref[...], preferred_element_type=_f32) + c_ref[...]
  hp = h * dinv_ref[...]
  h_ref[0] = hp[:, :32]
  h_ref[1] = hp[:, 32:]


def _m_tc(v, Weff, cshift, dinv):
  return pl.pallas_call(
      _m_body,
      grid=(GRID,),
      in_specs=[
          pl.BlockSpec((RB, H), lambda i: (i, 0)),
          pl.BlockSpec((H, H), lambda i: (0, 0)),
          pl.BlockSpec((1, H), lambda i: (0, 0)),
          pl.BlockSpec((RB, 1), lambda i: (i, 0)),
      ],
      out_specs=pl.BlockSpec((NC, RB, 32), lambda i: (0, i, 0)),
      out_shape=jax.ShapeDtypeStruct((NC, N, 32), _f32),
  )(v, Weff, cshift, dinv)


def _p_body(acc_ref, h_ref, dinv_ref, b_ref, v_ref, cs_ref, cq_ref):
  a = jnp.concatenate([acc_ref[0] + h_ref[0], acc_ref[1] + h_ref[1]], axis=1)
  v = jnp.maximum(a * dinv_ref[...] + b_ref[...], 0.0)
  v_ref[...] = v

  @pl.when(pl.program_id(0) == 0)
  def _():
    cs_ref[...] = jnp.zeros_like(cs_ref)
    cq_ref[...] = jnp.zeros_like(cq_ref)

  cs_ref[...] += jnp.sum(v, axis=0, keepdims=True)
  cq_ref[...] += jnp.sum(v * v, axis=0, keepdims=True)


def _p_tc(acc, hpl, dinv, b_row):
  return pl.pallas_call(
      _p_body,
      grid=(GRID,),
      in_specs=[
          pl.BlockSpec((NC, RB, 32), lambda i: (0, i, 0)),
          pl.BlockSpec((NC, RB, 32), lambda i: (0, i, 0)),
          pl.BlockSpec((RB, 1), lambda i: (i, 0)),
          pl.BlockSpec((1, H), lambda i: (0, 0)),
      ],
      out_specs=[
          pl.BlockSpec((RB, H), lambda i: (i, 0)),
          pl.BlockSpec((1, H), lambda i: (0, 0)),
          pl.BlockSpec((1, H), lambda i: (0, 0)),
      ],
      out_shape=[
          jax.ShapeDtypeStruct((N, H), _f32),
          jax.ShapeDtypeStruct((1, H), _f32),
          jax.ShapeDtypeStruct((1, H), _f32),
      ],
  )(acc, hpl, dinv, b_row)


def _p4_body(acc_ref, h_ref, dinv_ref, b_ref, bt_ref, g_ref, be_ref, fw_ref,
             fb_ref, cs_ref, cq_ref, seg_ref, cnt_ref, out_ref):
  i = pl.program_id(0)

  @pl.when(i < GRID)
  def _():
    a = jnp.concatenate([acc_ref[0] + h_ref[0], acc_ref[1] + h_ref[1]], axis=1)
    v = jnp.maximum(a * dinv_ref[...] + b_ref[...], 0.0)
    bt = bt_ref[0, 0, :]
    oh = (bt[:, None] == lax.broadcasted_iota(jnp.int32,
                                              (RB, B), 1)).astype(_f32)

    @pl.when(i == 0)
    def _():
      cs_ref[...] = jnp.zeros_like(cs_ref)
      cq_ref[...] = jnp.zeros_like(cq_ref)
      seg_ref[...] = jnp.zeros_like(seg_ref)
      cnt_ref[...] = jnp.zeros_like(cnt_ref)

    cs_ref[...] += jnp.sum(v, axis=0, keepdims=True)
    cq_ref[...] += jnp.sum(v * v, axis=0, keepdims=True)
    seg_ref[...] += lax.dot_general(oh, v, (((0,), (0,)), ((), ())),
                                    preferred_element_type=_f32)
    cnt_ref[...] += jnp.sum(oh, axis=0)[:, None]

  @pl.when(i == GRID)
  def _():
    mu = cs_ref[...] / N
    var = cq_ref[...] / N - mu * mu
    scale = g_ref[...] / jnp.sqrt(var + 1e-5)
    pooled = seg_ref[...] / jnp.maximum(cnt_ref[...], 1.0)
    pbn = scale * (pooled - mu) + be_ref[...]
    logits = (jnp.dot(pbn, fw_ref[...], preferred_element_type=_f32)
              + fb_ref[...])
    m = jnp.max(logits, axis=1, keepdims=True)
    e = jnp.exp(logits - m)
    out_ref[...] = logits - m - jnp.log(jnp.sum(e, axis=1, keepdims=True))


def _p4_tc(acc, hpl, dinv, b_row, batch3, g_row, be_row, fcW, fcb_row):
  blk = lambda i: (jnp.minimum(i, GRID - 1), 0)
  pblk = lambda i: (0, jnp.minimum(i, GRID - 1), 0)
  outs = pl.pallas_call(
      _p4_body,
      grid=(GRID + 1,),
      in_specs=[
          pl.BlockSpec((NC, RB, 32), pblk),
          pl.BlockSpec((NC, RB, 32), pblk),
          pl.BlockSpec((RB, 1), blk),
          pl.BlockSpec((1, H), lambda i: (0, 0)),
          pl.BlockSpec((1, 1, RB), lambda i: (jnp.minimum(i, GRID - 1), 0, 0)),
          pl.BlockSpec((1, H), lambda i: (0, 0)),
          pl.BlockSpec((1, H), lambda i: (0, 0)),
          pl.BlockSpec((H, C), lambda i: (0, 0)),
          pl.BlockSpec((1, C), lambda i: (0, 0)),
      ],
      out_specs=[
          pl.BlockSpec((1, H), lambda i: (0, 0)),
          pl.BlockSpec((1, H), lambda i: (0, 0)),
          pl.BlockSpec((B, H), lambda i: (0, 0)),
          pl.BlockSpec((B, 1), lambda i: (0, 0)),
          pl.BlockSpec((B, C), lambda i: (0, 0)),
      ],
      out_shape=[
          jax.ShapeDtypeStruct((1, H), _f32),
          jax.ShapeDtypeStruct((1, H), _f32),
          jax.ShapeDtypeStruct((B, H), _f32),
          jax.ShapeDtypeStruct((B, 1), _f32),
          jax.ShapeDtypeStruct((B, C), _f32),
      ],
  )(acc, hpl, dinv, b_row, batch3, g_row, be_row, fcW, fcb_row)
  return outs[4]


def _fold_bn(cs, cq, g, be, Wn):
  mu = cs[0] / N
  var = cq[0] / N - mu * mu
  scale = g / jnp.sqrt(var + 1e-5)
  shift = be - mu * scale
  return scale[:, None] * Wn, (shift @ Wn)[None, :]


def kernel(x, edge_index, batch, W1, b1, g1, be1, W2, b2, g2, be2, W3, b3, g3,
           be3, W4, b4, g4, be4, fcW, fcb):
  src = edge_index[0]
  dst = edge_index[1]
  padn = E_PAD - E
  src_p = jnp.concatenate([src, jnp.zeros((padn,), dtype=jnp.int32)])
  dst_p = jnp.concatenate(
      [dst, N + (jnp.arange(padn, dtype=jnp.int32) % (ACC_ROWS - N))])
  src_s = src_p.reshape(NS, CH_S, 128)
  dst_s = dst_p.reshape(NS, CH_S, 128)
  dst_d = dst_p.reshape(NC * NS, CH_D, 128)

  zrows32 = jnp.zeros((SPAN, 32), _f32)
  zrowsd = jnp.zeros((SPAN, DEGW), _f32)
  ones_rows = jnp.ones((128, DEGW), _f32)
  batch3 = batch.reshape(GRID, 1, RB)

  degp = _deg_sc(dst_d, ones_rows, zrowsd)

  hpl, dinv = _m1_tc(x, W1, degp)
  acc = _agg_sc(hpl, src_s, dst_s, zrows32)
  v, cs, cq = _p_tc(acc, hpl, dinv, b1.reshape(1, H))

  for (g_l, be_l, W_n, b_n) in ((g1, be1, W2, b2), (g2, be2, W3, b3),
                                (g3, be3, W4, b4)):
    Weff, cshift = _fold_bn(cs, cq, g_l, be_l, W_n)
    hpl = _m_tc(v, Weff, cshift, dinv)
    acc = _agg_sc(hpl, src_s, dst_s, zrows32)
    if W_n is W4:
      return _p4_tc(acc, hpl, dinv, b_n.reshape(1, H), batch3,
                    g4.reshape(1, H), be4.reshape(1, H), fcW,
                    fcb.reshape(1, C))
    v, cs, cq = _p_tc(acc, hpl, dinv, b_n.reshape(1, H))
